# Initial kernel scaffold; baseline (speedup 1.0000x reference)
import jax
import jax.numpy as jnp
from jax.experimental import pallas as pl


def kernel(pred, target):
    raise NotImplementedError("write your pallas kernel here")


# speedup vs baseline: 6.5168x; 6.5168x over previous
import functools

import jax
import jax.numpy as jnp
from jax.experimental import pallas as pl
from jax.experimental.pallas import tpu as pltpu

_ALPHA = 0.95
_LAMBDA = 0.5

_ROWS = 8192
_COLS = 128
_CHUNK = 256


def _cvar_body(k, n, pred_ref, tgt_ref, out_ref, err_ref):
    nchunks = _ROWS // _CHUNK

    def phase_a(c, acc_sq):
        sl = pl.ds(c * _CHUNK, _CHUNK)
        e = jnp.abs(pred_ref[sl, :] - tgt_ref[sl, :])
        err_ref[sl, :] = e
        return acc_sq + jnp.sum(e * e)

    sum_sq = jax.lax.fori_loop(0, nchunks, phase_a, jnp.float32(0.0))

    def count_ge(cand):
        def body(c, acc):
            sl = pl.ds(c * _CHUNK, _CHUNK)
            bits = jax.lax.bitcast_convert_type(err_ref[sl, :], jnp.int32)
            return acc + jnp.sum((bits >= cand).astype(jnp.int32))

        return jax.lax.fori_loop(0, nchunks, body, jnp.int32(0))

    def phase_b(i, prefix):
        cand = prefix | (jnp.int32(1) << (jnp.int32(30) - i))
        cnt = count_ge(cand)
        return jnp.where(cnt >= k, cand, prefix)

    t_bits = jax.lax.fori_loop(0, 31, phase_b, jnp.int32(0))

    def phase_c(c, carry):
        t_max, cnt_gt, s_gt = carry
        sl = pl.ds(c * _CHUNK, _CHUNK)
        e = err_ref[sl, :]
        bits = jax.lax.bitcast_convert_type(e, jnp.int32)
        gt = bits > t_bits
        t_max = jnp.maximum(t_max, jnp.max(jnp.where(bits == t_bits, e, 0.0)))
        cnt_gt = cnt_gt + jnp.sum(gt.astype(jnp.int32))
        s_gt = s_gt + jnp.sum(jnp.where(gt, e, 0.0))
        return t_max, cnt_gt, s_gt

    t_val, cnt_gt, s_gt = jax.lax.fori_loop(
        0, nchunks, phase_c, (jnp.float32(0.0), jnp.int32(0), jnp.float32(0.0))
    )

    cvar = (s_gt + (jnp.float32(k) - cnt_gt.astype(jnp.float32)) * t_val) / jnp.float32(k)
    mse = sum_sq / jnp.float32(n)
    out = (1.0 - _LAMBDA) * mse + _LAMBDA * cvar
    out_ref[:, :] = jnp.full((8, 128), out, jnp.float32)


@jax.jit
def kernel(pred, target):
    n = pred.shape[0]
    k = int(n * _ALPHA)
    p2 = pred.reshape(_ROWS, _COLS)
    t2 = target.reshape(_ROWS, _COLS)
    out = pl.pallas_call(
        functools.partial(_cvar_body, k, n),
        out_shape=jax.ShapeDtypeStruct((8, 128), jnp.float32),
        in_specs=[
            pl.BlockSpec((_ROWS, _COLS), lambda: (0, 0)),
            pl.BlockSpec((_ROWS, _COLS), lambda: (0, 0)),
        ],
        out_specs=pl.BlockSpec((8, 128), lambda: (0, 0)),
        scratch_shapes=[pltpu.VMEM((_ROWS, _COLS), jnp.float32)],
    )(p2, t2)
    return out[0, 0]
